# baseline (speedup 1.0000x reference)
import functools
import math

import jax
import jax.numpy as jnp
from jax import lax
from jax.experimental import pallas as pl
from jax.experimental.pallas import tpu as pltpu
from jax.experimental.pallas import tpu_sc as plsc

D_MODEL = 64
SCALE = math.sqrt(D_MODEL)

_INFO = plsc.get_sparse_core_info()
NUM_WORKERS = _INFO.num_cores * _INFO.num_subcores
LANES = _INFO.num_lanes

CHUNK = 128
NBUF = 2


def _make_sc_lookup(batch: int):
    assert batch % (NUM_WORKERS * CHUNK) == 0
    b_per_w = batch // NUM_WORKERS
    n_chunks = b_per_w // CHUNK

    mesh = plsc.VectorSubcoreMesh(core_axis_name="c", subcore_axis_name="s")

    @functools.partial(
        pl.kernel,
        out_type=jax.ShapeDtypeStruct((batch, D_MODEL), jnp.float32),
        mesh=mesh,
        compiler_params=pltpu.CompilerParams(use_tc_tiling_on_sc=False),
        scratch_types=[
            pltpu.VMEM((b_per_w,), jnp.int32),
            [pltpu.VMEM((CHUNK, D_MODEL), jnp.float32) for _ in range(NBUF)],
            [pltpu.SemaphoreType.DMA for _ in range(NBUF)],
        ],
    )
    def lookup(idx_hbm, table_hbm, out_hbm, idx_v, rows, sems):
        wid = lax.axis_index("s") * _INFO.num_cores + lax.axis_index("c")
        base = wid * b_per_w

        pltpu.sync_copy(idx_hbm.at[pl.ds(base, b_per_w)], idx_v)

        def start_gather(c, b):
            pltpu.async_copy(
                table_hbm.at[idx_v.at[pl.ds(c * CHUNK, CHUNK)]], rows[b], sems[b]
            )

        for b in range(NBUF):
            start_gather(b, b)

        @pl.loop(0, n_chunks, step=NBUF)
        def _chunk_loop(g):
            for b in range(NBUF):
                c = g + b
                pltpu.make_async_copy(
                    table_hbm.at[idx_v.at[pl.ds(c * CHUNK, CHUNK)]], rows[b], sems[b]
                ).wait()

                @pl.loop(0, CHUNK)
                def _scale_rows(r):
                    for j in range(D_MODEL // LANES):
                        sl = pl.ds(j * LANES, LANES)
                        rows[b][r, sl] = rows[b][r, sl] * SCALE

                pltpu.sync_copy(
                    rows[b], out_hbm.at[pl.ds(base + c * CHUNK, CHUNK)]
                )

                nxt = c + NBUF

                @pl.when(nxt < n_chunks)
                def _():
                    start_gather(nxt, b)

    return lookup


def kernel(x, table):
    batch = x.shape[0] * x.shape[1]
    flat_idx = x.reshape(batch)
    out = _make_sc_lookup(batch)(flat_idx, table)
    return out.reshape(x.shape[0], x.shape[1], D_MODEL)


# speedup vs baseline: 1.0565x; 1.0565x over previous
import functools
import math

import jax
import jax.numpy as jnp
from jax import lax
from jax.experimental import pallas as pl
from jax.experimental.pallas import tpu as pltpu
from jax.experimental.pallas import tpu_sc as plsc

D_MODEL = 64
SCALE = math.sqrt(D_MODEL)

_INFO = plsc.get_sparse_core_info()
NUM_WORKERS = _INFO.num_cores * _INFO.num_subcores
LANES = _INFO.num_lanes

CHUNK = 128
NBUF = 4


def _make_sc_lookup(batch: int):
    assert batch % (NUM_WORKERS * CHUNK) == 0
    b_per_w = batch // NUM_WORKERS
    n_chunks = b_per_w // CHUNK

    mesh = plsc.VectorSubcoreMesh(core_axis_name="c", subcore_axis_name="s")

    @functools.partial(
        pl.kernel,
        out_type=jax.ShapeDtypeStruct((batch, D_MODEL), jnp.float32),
        mesh=mesh,
        compiler_params=pltpu.CompilerParams(use_tc_tiling_on_sc=False),
        scratch_types=[
            pltpu.VMEM((b_per_w,), jnp.int32),
            [pltpu.VMEM((CHUNK, D_MODEL), jnp.float32) for _ in range(NBUF)],
            [pltpu.SemaphoreType.DMA for _ in range(NBUF)],
            [pltpu.SemaphoreType.DMA for _ in range(NBUF)],
        ],
    )
    def lookup(idx_hbm, table_hbm, out_hbm, idx_v, rows, g_sems, o_sems):
        wid = lax.axis_index("s") * _INFO.num_cores + lax.axis_index("c")
        base = wid * b_per_w

        pltpu.sync_copy(idx_hbm.at[pl.ds(base, b_per_w)], idx_v)

        def start_gather(c, b):
            pltpu.async_copy(
                table_hbm.at[idx_v.at[pl.ds(c * CHUNK, CHUNK)]], rows[b], g_sems[b]
            )

        def out_copy(c, b):
            return pltpu.make_async_copy(
                rows[b], out_hbm.at[pl.ds(base + c * CHUNK, CHUNK)], o_sems[b]
            )

        for b in range(NBUF):
            start_gather(b, b)

        @pl.loop(0, n_chunks, step=NBUF)
        def _chunk_loop(g):
            for b in range(NBUF):
                c = g + b
                pltpu.make_async_copy(
                    table_hbm.at[idx_v.at[pl.ds(c * CHUNK, CHUNK)]], rows[b], g_sems[b]
                ).wait()

                @pl.loop(0, CHUNK, unroll=8)
                def _scale_rows(r):
                    for j in range(D_MODEL // LANES):
                        sl = pl.ds(j * LANES, LANES)
                        rows[b][r, sl] = rows[b][r, sl] * SCALE

                out_copy(c, b).start()

                nxt = c + NBUF

                @pl.when(nxt < n_chunks)
                def _():
                    out_copy(c, b).wait()
                    start_gather(nxt, b)

        for b in range(NBUF):
            out_copy(n_chunks - NBUF + b, b).wait()

    return lookup


def kernel(x, table):
    batch = x.shape[0] * x.shape[1]
    flat_idx = x.reshape(batch)
    out = _make_sc_lookup(batch)(flat_idx, table)
    return out.reshape(x.shape[0], x.shape[1], D_MODEL)
